# baseline (speedup 1.0000x reference)
import functools

import jax
import jax.numpy as jnp
from jax import lax
from jax.experimental import pallas as pl
from jax.experimental.pallas import tpu as pltpu
from jax.experimental.pallas import tpu_sc as plsc

_N = 4096
_D = 512
_BN = 512
_NW = 32
_RC = 8
_UNROLL = 4


def _sc_s_body(adj_hbm, x0_hbm, s_hbm, x0_v, buf0, buf1, s_v, acc_m,
               sem0, sem1):
    cid = lax.axis_index("c")
    sid = lax.axis_index("s")
    wid = sid * 2 + cid
    rows = _N // _NW
    base = wid * rows
    pltpu.sync_copy(x0_hbm, x0_v)
    nch = rows // _RC
    bufs = (buf0, buf1)
    sems = (sem0, sem1)
    pltpu.make_async_copy(
        adj_hbm.at[pl.ds(base, _RC), :], buf0, sem0).start()
    lanes = lax.iota(jnp.int32, 16)
    pending = []
    for ch in range(nch):
        buf = bufs[ch % 2]
        pltpu.make_async_copy(
            adj_hbm.at[pl.ds(base + ch * _RC, _RC), :],
            buf, sems[ch % 2]).wait()
        if ch + 1 < nch:
            pltpu.make_async_copy(
                adj_hbm.at[pl.ds(base + (ch + 1) * _RC, _RC), :],
                bufs[(ch + 1) % 2], sems[(ch + 1) % 2]).start()

        def col_body(i, accs, buf=buf):
            accs = list(accs)
            for u in range(_UNROLL):
                cc = i * _UNROLL + u
                x = x0_v[pl.ds(cc * 16, 16)]
                for r in range(_RC):
                    a = buf[r, pl.ds(cc * 16, 16)]
                    accs[r] = accs[r] + a.astype(jnp.float32) * x
            return tuple(accs)

        init = tuple(jnp.zeros((16,), jnp.float32) for _ in range(_RC))
        accs = lax.fori_loop(0, _N // (16 * _UNROLL), col_body, init)
        pending.extend(accs)
        if len(pending) == 16:
            for r, acc in enumerate(pending):
                acc_m[r, :] = acc
            total = jnp.zeros((16,), jnp.float32)
            for j in range(16):
                col = plsc.load_gather(
                    acc_m, [lanes, jnp.full((16,), j, jnp.int32)])
                total = total + col
            s_v[pl.ds((ch // 2) * 16, 16)] = total
            pending = []
    pltpu.sync_copy(s_v, s_hbm.at[pl.ds(base, rows)])


@functools.partial(
    pl.kernel,
    out_type=jax.ShapeDtypeStruct((_N,), jnp.float32),
    mesh=plsc.VectorSubcoreMesh(core_axis_name="c", subcore_axis_name="s"),
    scratch_types=[
        pltpu.VMEM((_N,), jnp.float32),
        pltpu.VMEM((_RC, _N), jnp.int32),
        pltpu.VMEM((_RC, _N), jnp.int32),
        pltpu.VMEM((_N // _NW,), jnp.float32),
        pltpu.VMEM((16, 16), jnp.float32),
        pltpu.SemaphoreType.DMA,
        pltpu.SemaphoreType.DMA,
    ],
    compiler_params=pltpu.CompilerParams(needs_layout_passes=False),
)
def _sc_s(adj_hbm, x0_hbm, s_hbm, x0_v, buf0, buf1, s_v, acc_m, sem0, sem1):
    _sc_s_body(adj_hbm, x0_hbm, s_hbm, x0_v, buf0, buf1, s_v, acc_m, sem0, sem1)


def _tc_body(s_ref, nf_ref, ws_ref, wn_ref, out_ref):
    s = s_ref[...]
    w = jnp.sum(wn_ref[...], axis=1, keepdims=True)
    h = lax.dot_general(nf_ref[...], ws_ref[...],
                        (((1,), (1,)), ((), ())),
                        preferred_element_type=jnp.float32)
    neigh = lax.dot_general(s, w, (((1,), (1,)), ((), ())),
                            preferred_element_type=jnp.float32)
    out_ref[...] = jnp.maximum(h + neigh, 0.0)


@jax.jit
def kernel(node_feats, adj_matrix, W_self, W_neigh):
    n, d = node_feats.shape
    x0 = node_feats[:, 0]
    s = _sc_s(adj_matrix, x0)
    grid = (n // _BN,)
    return pl.pallas_call(
        _tc_body,
        grid=grid,
        in_specs=[
            pl.BlockSpec((_BN, 1), lambda i: (i, 0)),
            pl.BlockSpec((_BN, d), lambda i: (i, 0)),
            pl.BlockSpec((d, d), lambda i: (0, 0)),
            pl.BlockSpec((d, d), lambda i: (0, 0)),
        ],
        out_specs=pl.BlockSpec((_BN, d), lambda i: (i, 0)),
        out_shape=jax.ShapeDtypeStruct((n, d), jnp.float32),
        compiler_params=pltpu.CompilerParams(
            dimension_semantics=("arbitrary",),
        ),
    )(s.reshape(n, 1), node_feats, W_self, W_neigh)


# speedup vs baseline: 1.2850x; 1.2850x over previous
import functools

import jax
import jax.numpy as jnp
from jax import lax
from jax.experimental import pallas as pl
from jax.experimental.pallas import tpu as pltpu
from jax.experimental.pallas import tpu_sc as plsc

_N = 4096
_D = 512
_BN = 512
_R_SC = 2048
_NW = 32
_RC = 8


def _sc_s_body(adj_hbm, x0_hbm, s_hbm, x0_v, buf0, buf1, s_v, acc_m,
               sem0, sem1):
    cid = lax.axis_index("c")
    sid = lax.axis_index("s")
    wid = sid * 2 + cid
    rows = _R_SC // _NW
    base = wid * rows
    pltpu.sync_copy(x0_hbm, x0_v)
    nch = rows // _RC
    bufs = (buf0, buf1)
    sems = (sem0, sem1)
    pltpu.make_async_copy(
        adj_hbm.at[pl.ds(base, _RC), :], buf0, sem0).start()
    lanes = lax.iota(jnp.int32, 16)
    pending = []
    for ch in range(nch):
        buf = bufs[ch % 2]
        pltpu.make_async_copy(
            adj_hbm.at[pl.ds(base + ch * _RC, _RC), :],
            buf, sems[ch % 2]).wait()
        if ch + 1 < nch:
            pltpu.make_async_copy(
                adj_hbm.at[pl.ds(base + (ch + 1) * _RC, _RC), :],
                bufs[(ch + 1) % 2], sems[(ch + 1) % 2]).start()

        init = tuple(jnp.zeros((16,), jnp.float32) for _ in range(_RC))

        @plsc.parallel_loop(0, _N // 16, 1, unroll=8, carry=init)
        def col_body(c, accs, buf=buf):
            x = x0_v[pl.ds(c * 16, 16)]
            return tuple(
                accs[r] + buf[r, pl.ds(c * 16, 16)].astype(jnp.float32) * x
                for r in range(_RC))

        pending.extend(col_body)
        if len(pending) == 16:
            for r, acc in enumerate(pending):
                acc_m[r, :] = acc
            total = jnp.zeros((16,), jnp.float32)
            for j in range(16):
                col = plsc.load_gather(
                    acc_m, [lanes, jnp.full((16,), j, jnp.int32)])
                total = total + col
            s_v[pl.ds((ch // 2) * 16, 16)] = total
            pending = []
    pltpu.sync_copy(s_v, s_hbm.at[pl.ds(base, rows)])


@functools.partial(
    pl.kernel,
    out_type=jax.ShapeDtypeStruct((_R_SC,), jnp.float32),
    mesh=plsc.VectorSubcoreMesh(core_axis_name="c", subcore_axis_name="s"),
    scratch_types=[
        pltpu.VMEM((_N,), jnp.float32),
        pltpu.VMEM((_RC, _N), jnp.int32),
        pltpu.VMEM((_RC, _N), jnp.int32),
        pltpu.VMEM((_R_SC // _NW,), jnp.float32),
        pltpu.VMEM((16, 16), jnp.float32),
        pltpu.SemaphoreType.DMA,
        pltpu.SemaphoreType.DMA,
    ],
    compiler_params=pltpu.CompilerParams(needs_layout_passes=False),
)
def _sc_s(adj_hbm, x0_hbm, s_hbm, x0_v, buf0, buf1, s_v, acc_m, sem0, sem1):
    _sc_s_body(adj_hbm, x0_hbm, s_hbm, x0_v, buf0, buf1, s_v, acc_m,
               sem0, sem1)


def _tc_hi_body(x0_ref, nf_ref, adj_ref, ws_ref, wn_ref, out_ref):
    a = adj_ref[...]
    x0 = x0_ref[...]
    s = jnp.sum(a.astype(jnp.float32) * x0, axis=1, keepdims=True)
    w = jnp.sum(wn_ref[...], axis=1, keepdims=True)
    h = lax.dot_general(nf_ref[...], ws_ref[...],
                        (((1,), (1,)), ((), ())),
                        preferred_element_type=jnp.float32)
    neigh = lax.dot_general(s, w, (((1,), (1,)), ((), ())),
                            preferred_element_type=jnp.float32)
    out_ref[...] = jnp.maximum(h + neigh, 0.0)


def _tc_lo_body(prev_ref, s_ref, nf_ref, ws_ref, wn_ref, out_ref):
    del prev_ref
    s = s_ref[...]
    w = jnp.sum(wn_ref[...], axis=1, keepdims=True)
    h = lax.dot_general(nf_ref[...], ws_ref[...],
                        (((1,), (1,)), ((), ())),
                        preferred_element_type=jnp.float32)
    neigh = lax.dot_general(s, w, (((1,), (1,)), ((), ())),
                            preferred_element_type=jnp.float32)
    out_ref[...] = jnp.maximum(h + neigh, 0.0)


@jax.jit
def kernel(node_feats, adj_matrix, W_self, W_neigh):
    n, d = node_feats.shape
    x0 = node_feats[:, 0]
    s_lo = _sc_s(adj_matrix, x0)
    x0_row = x0.reshape(1, n)
    hi_blocks = (n - _R_SC) // _BN
    lo_blocks = _R_SC // _BN
    off = _R_SC // _BN

    out_hi = pl.pallas_call(
        _tc_hi_body,
        grid=(hi_blocks,),
        in_specs=[
            pl.BlockSpec((1, n), lambda i: (0, 0)),
            pl.BlockSpec((_BN, d), lambda i: (i + off, 0)),
            pl.BlockSpec((_BN, n), lambda i: (i + off, 0)),
            pl.BlockSpec((d, d), lambda i: (0, 0)),
            pl.BlockSpec((d, d), lambda i: (0, 0)),
        ],
        out_specs=pl.BlockSpec((_BN, d), lambda i: (i + off, 0)),
        out_shape=jax.ShapeDtypeStruct((n, d), jnp.float32),
        compiler_params=pltpu.CompilerParams(
            dimension_semantics=("arbitrary",),
        ),
    )(x0_row, node_feats, adj_matrix, W_self, W_neigh)

    return pl.pallas_call(
        _tc_lo_body,
        grid=(lo_blocks,),
        in_specs=[
            pl.BlockSpec(memory_space=pl.ANY),
            pl.BlockSpec((_BN, 1), lambda i: (i, 0)),
            pl.BlockSpec((_BN, d), lambda i: (i, 0)),
            pl.BlockSpec((d, d), lambda i: (0, 0)),
            pl.BlockSpec((d, d), lambda i: (0, 0)),
        ],
        out_specs=pl.BlockSpec((_BN, d), lambda i: (i, 0)),
        out_shape=jax.ShapeDtypeStruct((n, d), jnp.float32),
        input_output_aliases={0: 0},
        compiler_params=pltpu.CompilerParams(
            dimension_semantics=("arbitrary",),
        ),
    )(out_hi, s_lo.reshape(_R_SC, 1), node_feats, W_self, W_neigh)


# speedup vs baseline: 1.3204x; 1.0276x over previous
import functools

import jax
import jax.numpy as jnp
from jax import lax
from jax.experimental import pallas as pl
from jax.experimental.pallas import tpu as pltpu
from jax.experimental.pallas import tpu_sc as plsc

_N = 4096
_D = 512
_BN = 512
_R_SC = 2048
_NW = 32
_RC = 8


def _sc_s_body(adj_hbm, x0_hbm, s_hbm, x0_v, buf0, buf1, s_v, acc_m,
               sem0, sem1):
    cid = lax.axis_index("c")
    sid = lax.axis_index("s")
    wid = sid * 2 + cid
    rows = _R_SC // _NW
    base = wid * rows
    pltpu.sync_copy(x0_hbm, x0_v)
    nch = rows // _RC
    bufs = (buf0, buf1)
    sems = (sem0, sem1)
    pltpu.make_async_copy(
        adj_hbm.at[pl.ds(base, _RC), :], buf0, sem0).start()
    pltpu.make_async_copy(
        adj_hbm.at[pl.ds(base + _RC, _RC), :], buf1, sem1).start()
    lanes = lax.iota(jnp.int32, 16)

    def pair_body(g, _):
        for half in range(2):
            ch = g * 2 + half
            buf, sem = bufs[half], sems[half]
            pltpu.make_async_copy(
                adj_hbm.at[pl.ds(base + ch * _RC, _RC), :],
                buf, sem).wait()

            init = tuple(jnp.zeros((16,), jnp.float32) for _ in range(_RC))

            @plsc.parallel_loop(0, _N // 16, 1, unroll=4, carry=init)
            def col_body(c, accs, buf=buf):
                x = x0_v[pl.ds(c * 16, 16)]
                return tuple(
                    accs[r]
                    + buf[r, pl.ds(c * 16, 16)].astype(jnp.float32) * x
                    for r in range(_RC))

            for r in range(_RC):
                acc_m[half * _RC + r, :] = col_body[r]

            @pl.when(ch + 2 < nch)
            def _():
                pltpu.make_async_copy(
                    adj_hbm.at[pl.ds(base + (ch + 2) * _RC, _RC), :],
                    buf, sem).start()

        total = jnp.zeros((16,), jnp.float32)
        for j in range(16):
            col = plsc.load_gather(
                acc_m, [lanes, jnp.full((16,), j, jnp.int32)])
            total = total + col
        s_v[pl.ds(g * 16, 16)] = total
        return 0

    lax.fori_loop(0, nch // 2, pair_body, 0)
    pltpu.sync_copy(s_v, s_hbm.at[pl.ds(base, rows)])


@functools.partial(
    pl.kernel,
    out_type=jax.ShapeDtypeStruct((_R_SC,), jnp.float32),
    mesh=plsc.VectorSubcoreMesh(core_axis_name="c", subcore_axis_name="s"),
    scratch_types=[
        pltpu.VMEM((_N,), jnp.float32),
        pltpu.VMEM((_RC, _N), jnp.int32),
        pltpu.VMEM((_RC, _N), jnp.int32),
        pltpu.VMEM((_R_SC // _NW,), jnp.float32),
        pltpu.VMEM((16, 16), jnp.float32),
        pltpu.SemaphoreType.DMA,
        pltpu.SemaphoreType.DMA,
    ],
    compiler_params=pltpu.CompilerParams(needs_layout_passes=False),
)
def _sc_s(adj_hbm, x0_hbm, s_hbm, x0_v, buf0, buf1, s_v, acc_m, sem0, sem1):
    _sc_s_body(adj_hbm, x0_hbm, s_hbm, x0_v, buf0, buf1, s_v, acc_m,
               sem0, sem1)


def _tc_hi_body(x0_ref, nf_ref, adj_ref, ws_ref, wn_ref, out_ref):
    a = adj_ref[...]
    x0 = x0_ref[...]
    s = jnp.sum(a.astype(jnp.float32) * x0, axis=1, keepdims=True)
    w = jnp.sum(wn_ref[...], axis=1, keepdims=True)
    h = lax.dot_general(nf_ref[...], ws_ref[...],
                        (((1,), (1,)), ((), ())),
                        preferred_element_type=jnp.float32)
    neigh = lax.dot_general(s, w, (((1,), (1,)), ((), ())),
                            preferred_element_type=jnp.float32)
    out_ref[...] = jnp.maximum(h + neigh, 0.0)


def _tc_lo_body(prev_ref, s_ref, nf_ref, ws_ref, wn_ref, out_ref):
    del prev_ref
    s = s_ref[...]
    w = jnp.sum(wn_ref[...], axis=1, keepdims=True)
    h = lax.dot_general(nf_ref[...], ws_ref[...],
                        (((1,), (1,)), ((), ())),
                        preferred_element_type=jnp.float32)
    neigh = lax.dot_general(s, w, (((1,), (1,)), ((), ())),
                            preferred_element_type=jnp.float32)
    out_ref[...] = jnp.maximum(h + neigh, 0.0)


@jax.jit
def kernel(node_feats, adj_matrix, W_self, W_neigh):
    n, d = node_feats.shape
    x0 = node_feats[:, 0]
    s_lo = _sc_s(adj_matrix, x0)
    x0_row = x0.reshape(1, n)
    hi_blocks = (n - _R_SC) // _BN
    lo_blocks = _R_SC // _BN
    off = _R_SC // _BN

    out_hi = pl.pallas_call(
        _tc_hi_body,
        grid=(hi_blocks,),
        in_specs=[
            pl.BlockSpec((1, n), lambda i: (0, 0)),
            pl.BlockSpec((_BN, d), lambda i: (i + off, 0)),
            pl.BlockSpec((_BN, n), lambda i: (i + off, 0)),
            pl.BlockSpec((d, d), lambda i: (0, 0)),
            pl.BlockSpec((d, d), lambda i: (0, 0)),
        ],
        out_specs=pl.BlockSpec((_BN, d), lambda i: (i + off, 0)),
        out_shape=jax.ShapeDtypeStruct((n, d), jnp.float32),
        compiler_params=pltpu.CompilerParams(
            dimension_semantics=("arbitrary",),
        ),
    )(x0_row, node_feats, adj_matrix, W_self, W_neigh)

    return pl.pallas_call(
        _tc_lo_body,
        grid=(lo_blocks,),
        in_specs=[
            pl.BlockSpec(memory_space=pl.ANY),
            pl.BlockSpec((_BN, 1), lambda i: (i, 0)),
            pl.BlockSpec((_BN, d), lambda i: (i, 0)),
            pl.BlockSpec((d, d), lambda i: (0, 0)),
            pl.BlockSpec((d, d), lambda i: (0, 0)),
        ],
        out_specs=pl.BlockSpec((_BN, d), lambda i: (i, 0)),
        out_shape=jax.ShapeDtypeStruct((n, d), jnp.float32),
        input_output_aliases={0: 0},
        compiler_params=pltpu.CompilerParams(
            dimension_semantics=("arbitrary",),
        ),
    )(out_hi, s_lo.reshape(_R_SC, 1), node_feats, W_self, W_neigh)


# speedup vs baseline: 2.2429x; 1.6987x over previous
import jax
import jax.numpy as jnp
from jax import lax
from jax.experimental import pallas as pl
from jax.experimental.pallas import tpu as pltpu

_BN = 512


def _body(nfc_ref, nf_ref, adj_ref, ws_ref, wn_ref, out_ref):
    a = adj_ref[...]
    xc = nfc_ref[...].astype(jnp.bfloat16)
    s_wide = lax.dot_general(a.astype(jnp.bfloat16), xc,
                             (((1,), (0,)), ((), ())),
                             preferred_element_type=jnp.float32)
    s = s_wide[:, 0:1]
    w = jnp.sum(wn_ref[...], axis=1, keepdims=True)
    h = lax.dot_general(nf_ref[...], ws_ref[...],
                        (((1,), (1,)), ((), ())),
                        preferred_element_type=jnp.float32)
    neigh = lax.dot_general(s, w, (((1,), (1,)), ((), ())),
                            preferred_element_type=jnp.float32)
    out_ref[...] = jnp.maximum(h + neigh, 0.0)


@jax.jit
def kernel(node_feats, adj_matrix, W_self, W_neigh):
    n, d = node_feats.shape
    grid = (n // _BN,)
    return pl.pallas_call(
        _body,
        grid=grid,
        in_specs=[
            pl.BlockSpec((n, 128), lambda i: (0, 0)),
            pl.BlockSpec((_BN, d), lambda i: (i, 0)),
            pl.BlockSpec((_BN, n), lambda i: (i, 0)),
            pl.BlockSpec((d, d), lambda i: (0, 0)),
            pl.BlockSpec((d, d), lambda i: (0, 0)),
        ],
        out_specs=pl.BlockSpec((_BN, d), lambda i: (i, 0)),
        out_shape=jax.ShapeDtypeStruct((n, d), jnp.float32),
        compiler_params=pltpu.CompilerParams(
            dimension_semantics=("arbitrary",),
        ),
    )(node_feats, node_feats, adj_matrix, W_self, W_neigh)


# speedup vs baseline: 2.4518x; 1.0931x over previous
import jax
import jax.numpy as jnp
from jax import lax
from jax.experimental import pallas as pl
from jax.experimental.pallas import tpu as pltpu

_BN = 512


def _body(nfc_ref, nf_ref, adj_ref, ws_ref, wn_ref, out_ref, x0_ref):
    @pl.when(pl.program_id(0) == 0)
    def _():
        x0_ref[...] = nfc_ref[...][:, 0:1].T

    a = adj_ref[...]
    x0 = x0_ref[...]
    s = jnp.sum(a.astype(jnp.float32) * x0, axis=1, keepdims=True)
    w = jnp.sum(wn_ref[...], axis=1, keepdims=True)
    h = lax.dot_general(nf_ref[...], ws_ref[...],
                        (((1,), (1,)), ((), ())),
                        preferred_element_type=jnp.float32)
    neigh = lax.dot_general(s, w, (((1,), (1,)), ((), ())),
                            preferred_element_type=jnp.float32)
    out_ref[...] = jnp.maximum(h + neigh, 0.0)


@jax.jit
def kernel(node_feats, adj_matrix, W_self, W_neigh):
    n, d = node_feats.shape
    grid = (n // _BN,)
    return pl.pallas_call(
        _body,
        grid=grid,
        in_specs=[
            pl.BlockSpec((n, 128), lambda i: (0, 0)),
            pl.BlockSpec((_BN, d), lambda i: (i, 0)),
            pl.BlockSpec((_BN, n), lambda i: (i, 0)),
            pl.BlockSpec((d, d), lambda i: (0, 0)),
            pl.BlockSpec((d, d), lambda i: (0, 0)),
        ],
        out_specs=pl.BlockSpec((_BN, d), lambda i: (i, 0)),
        out_shape=jax.ShapeDtypeStruct((n, d), jnp.float32),
        scratch_shapes=[pltpu.VMEM((1, n), jnp.float32)],
        compiler_params=pltpu.CompilerParams(
            dimension_semantics=("arbitrary",),
        ),
    )(node_feats, node_feats, adj_matrix, W_self, W_neigh)


# speedup vs baseline: 2.4579x; 1.0025x over previous
import jax
import jax.numpy as jnp
from jax import lax
from jax.experimental import pallas as pl
from jax.experimental.pallas import tpu as pltpu

_BN = 512


def _body(nfc_ref, nf_ref, adj_lo_ref, adj_hi_ref, ws_ref, wn_ref, out_ref,
          x0_ref):
    @pl.when(pl.program_id(0) == 0)
    def _():
        x0_ref[...] = nfc_ref[...][:, 0:1].T

    n2 = adj_lo_ref.shape[1]
    a_lo = adj_lo_ref[...]
    a_hi = adj_hi_ref[...]
    x0 = x0_ref[...]
    s = (jnp.sum(a_lo.astype(jnp.float32) * x0[:, :n2],
                 axis=1, keepdims=True)
         + jnp.sum(a_hi.astype(jnp.float32) * x0[:, n2:],
                   axis=1, keepdims=True))
    w = jnp.sum(wn_ref[...], axis=1, keepdims=True)
    h = lax.dot_general(nf_ref[...], ws_ref[...],
                        (((1,), (1,)), ((), ())),
                        preferred_element_type=jnp.float32)
    neigh = lax.dot_general(s, w, (((1,), (1,)), ((), ())),
                            preferred_element_type=jnp.float32)
    out_ref[...] = jnp.maximum(h + neigh, 0.0)


@jax.jit
def kernel(node_feats, adj_matrix, W_self, W_neigh):
    n, d = node_feats.shape
    grid = (n // _BN,)
    return pl.pallas_call(
        _body,
        grid=grid,
        in_specs=[
            pl.BlockSpec((n, 128), lambda i: (0, 0)),
            pl.BlockSpec((_BN, d), lambda i: (i, 0)),
            pl.BlockSpec((_BN, n // 2), lambda i: (i, 0)),
            pl.BlockSpec((_BN, n // 2), lambda i: (i, 1)),
            pl.BlockSpec((d, d), lambda i: (0, 0)),
            pl.BlockSpec((d, d), lambda i: (0, 0)),
        ],
        out_specs=pl.BlockSpec((_BN, d), lambda i: (i, 0)),
        out_shape=jax.ShapeDtypeStruct((n, d), jnp.float32),
        scratch_shapes=[pltpu.VMEM((1, n), jnp.float32)],
        compiler_params=pltpu.CompilerParams(
            dimension_semantics=("arbitrary",),
        ),
    )(node_feats, node_feats, adj_matrix, adj_matrix, W_self, W_neigh)
